# baseline (speedup 1.0000x reference)
import functools

import jax
import jax.numpy as jnp
from jax import lax
from jax.experimental import pallas as pl
from jax.experimental.pallas import tpu as pltpu
from jax.experimental.pallas import tpu_sc as plsc

EMB = 16
N_EDGES = 1600000
CHUNK = 1280
NUM_CHUNKS = N_EDGES // CHUNK
NW = 32
L = 16
NBUF = 2


def _sc_lookup(edge_attr, Wc_rep):
    mesh = plsc.VectorSubcoreMesh(core_axis_name="c", subcore_axis_name="s")

    @functools.partial(
        pl.kernel,
        mesh=mesh,
        compiler_params=pltpu.CompilerParams(
            use_tc_tiling_on_sc=False, needs_layout_passes=False),
        out_type=jax.ShapeDtypeStruct((N_EDGES, 2 * EMB), jnp.float32),
        scratch_types=[
            [pltpu.VMEM((CHUNK, 2), jnp.int32) for _ in range(NBUF)],
            [pltpu.VMEM((CHUNK,), jnp.int32) for _ in range(NBUF)],
            [pltpu.VMEM((CHUNK, 2 * EMB), jnp.float32) for _ in range(NBUF)],
            [pltpu.SemaphoreType.DMA for _ in range(NBUF)],
            [pltpu.SemaphoreType.DMA for _ in range(NBUF)],
            [pltpu.SemaphoreType.DMA for _ in range(NBUF)],
        ],
    )
    def k(ea_hbm, wc_hbm, out_hbm,
          ea_v, ci_v, out_v, isem, gsem, wsem):
        wid = lax.axis_index("s") * 2 + lax.axis_index("c")
        steps = (NUM_CHUNKS + NW - 1) // NW
        tw = (NUM_CHUNKS - wid + NW - 1) // NW

        def start_idx(t, b):
            base = (wid + t * NW) * CHUNK
            pltpu.async_copy(
                ea_hbm.at[pl.ds(base, CHUNK), :], ea_v[b], isem[b])

        def wait_idx(b):
            pltpu.make_async_copy(
                ea_hbm.at[pl.ds(0, CHUNK), :], ea_v[b], isem[b]).wait()

        def wait_write(b):
            pltpu.make_async_copy(
                out_v[b], out_hbm.at[pl.ds(0, CHUNK), :], wsem[b]).wait()

        def run_chunk(t, b):
            wait_idx(b)
            lanes = lax.iota(jnp.int32, L)
            zeros = jnp.zeros((L,), jnp.int32)
            ones = jnp.ones((L,), jnp.int32)
            rep = wid * 16
            for o in range(0, CHUNK, L):
                rows = lanes + o
                i0 = plsc.load_gather(ea_v[b], [rows, zeros])
                i1 = plsc.load_gather(ea_v[b], [rows, ones])
                ci_v[b][pl.ds(o, L)] = i0 * 4 + i1 + rep
            cps = []
            for j in range(0, CHUNK, 128):
                cps.append(pltpu.async_copy(
                    wc_hbm.at[ci_v[b].at[pl.ds(j, 128)]],
                    out_v[b].at[pl.ds(j, 128), :], gsem[b]))
            for cp in cps:
                cp.wait()
            base = (wid + t * NW) * CHUNK
            pltpu.async_copy(out_v[b], out_hbm.at[pl.ds(base, CHUNK), :], wsem[b])

        start_idx(0, 0)

        def body(t, carry):
            for bb in range(NBUF):
                @pl.when(lax.rem(t, NBUF) == bb)
                def _(bb=bb):
                    @pl.when(t + 1 < tw)
                    def _():
                        start_idx(t + 1, (bb + 1) % NBUF)

                    @pl.when(t < tw)
                    def _():
                        @pl.when(t >= NBUF)
                        def _():
                            wait_write(bb)
                        run_chunk(t, bb)
            return carry

        lax.fori_loop(0, steps, body, 0)

        for kk in range(NBUF):
            tp = tw - 1 - kk
            for bb in range(NBUF):
                @pl.when(jnp.logical_and(tp >= 0, lax.rem(tp, NBUF) == bb))
                def _(bb=bb):
                    wait_write(bb)

    return k(edge_attr, Wc_rep)


def kernel(edge_attr, W0, W1):
    Wc = jnp.concatenate(
        [jnp.repeat(W0, 4, axis=0), jnp.tile(W1, (4, 1))], axis=1)
    Wc_rep = jnp.tile(Wc, (NW, 1))
    return _sc_lookup(edge_attr, Wc_rep)


# speedup vs baseline: 1.1460x; 1.1460x over previous
import functools

import jax
import jax.numpy as jnp
from jax import lax
from jax.experimental import pallas as pl
from jax.experimental.pallas import tpu as pltpu
from jax.experimental.pallas import tpu_sc as plsc

EMB = 16
N_EDGES = 1600000
CHUNK = 1280
NUM_CHUNKS = N_EDGES // CHUNK
NW = 32
L = 16
NBUF = 2


def _sc_lookup(edge_attr, Wc_rep):
    mesh = plsc.VectorSubcoreMesh(core_axis_name="c", subcore_axis_name="s")

    @functools.partial(
        pl.kernel,
        mesh=mesh,
        compiler_params=pltpu.CompilerParams(
            use_tc_tiling_on_sc=False, needs_layout_passes=False),
        out_type=jax.ShapeDtypeStruct((N_EDGES, 2 * EMB), jnp.float32),
        scratch_types=[
            [pltpu.VMEM((2 * CHUNK,), jnp.int32) for _ in range(NBUF)],
            [pltpu.VMEM((CHUNK,), jnp.int32) for _ in range(NBUF)],
            [pltpu.VMEM((CHUNK, 2 * EMB), jnp.float32) for _ in range(NBUF)],
            [pltpu.SemaphoreType.DMA for _ in range(NBUF)],
            [pltpu.SemaphoreType.DMA for _ in range(NBUF)],
            [pltpu.SemaphoreType.DMA for _ in range(NBUF)],
        ],
    )
    def k(ea_hbm, wc_hbm, out_hbm,
          ea_v, ci_v, out_v, isem, gsem, wsem):
        wid = lax.axis_index("s") * 2 + lax.axis_index("c")
        steps = (NUM_CHUNKS + NW - 1) // NW
        tw = (NUM_CHUNKS - wid + NW - 1) // NW

        def start_idx(t, b):
            base = (wid + t * NW) * CHUNK
            pltpu.async_copy(
                ea_hbm.at[pl.ds(2 * base, 2 * CHUNK)], ea_v[b], isem[b])

        def wait_idx(b):
            pltpu.make_async_copy(
                ea_hbm.at[pl.ds(0, 2 * CHUNK)], ea_v[b], isem[b]).wait()

        def wait_write(b):
            pltpu.make_async_copy(
                out_v[b], out_hbm.at[pl.ds(0, CHUNK), :], wsem[b]).wait()

        def run_chunk(t, b):
            wait_idx(b)
            lanes2 = lax.iota(jnp.int32, L) * 2
            rep = wid * 16
            for o in range(0, CHUNK, L):
                even = lanes2 + (2 * o)
                i0 = plsc.load_gather(ea_v[b], [even])
                i1 = plsc.load_gather(ea_v[b], [even + 1])
                ci_v[b][pl.ds(o, L)] = i0 * 4 + i1 + rep
            cps = []
            for j in range(0, CHUNK, 128):
                cps.append(pltpu.async_copy(
                    wc_hbm.at[ci_v[b].at[pl.ds(j, 128)]],
                    out_v[b].at[pl.ds(j, 128), :], gsem[b]))
            for cp in cps:
                cp.wait()
            base = (wid + t * NW) * CHUNK
            pltpu.async_copy(out_v[b], out_hbm.at[pl.ds(base, CHUNK), :], wsem[b])

        start_idx(0, 0)

        def body(t, carry):
            for bb in range(NBUF):
                @pl.when(lax.rem(t, NBUF) == bb)
                def _(bb=bb):
                    @pl.when(t + 1 < tw)
                    def _():
                        start_idx(t + 1, (bb + 1) % NBUF)

                    @pl.when(t < tw)
                    def _():
                        @pl.when(t >= NBUF)
                        def _():
                            wait_write(bb)
                        run_chunk(t, bb)
            return carry

        lax.fori_loop(0, steps, body, 0)

        for kk in range(NBUF):
            tp = tw - 1 - kk
            for bb in range(NBUF):
                @pl.when(jnp.logical_and(tp >= 0, lax.rem(tp, NBUF) == bb))
                def _(bb=bb):
                    wait_write(bb)

    return k(edge_attr, Wc_rep)


def kernel(edge_attr, W0, W1):
    Wc = jnp.concatenate(
        [jnp.repeat(W0, 4, axis=0), jnp.tile(W1, (4, 1))], axis=1)
    Wc_rep = jnp.tile(Wc, (NW, 1))
    ea_flat = edge_attr.reshape(-1)
    return _sc_lookup(ea_flat, Wc_rep)


# speedup vs baseline: 2.9553x; 2.5788x over previous
import functools

import jax
import jax.numpy as jnp
from jax import lax
from jax.experimental import pallas as pl
from jax.experimental.pallas import tpu as pltpu
from jax.experimental.pallas import tpu_sc as plsc

EMB = 16
N_EDGES = 1600000
CHUNK = 1280
NUM_CHUNKS = N_EDGES // CHUNK
NW = 32
NBUF = 2


def _sc_lookup(ci, Wc_rep):
    mesh = plsc.VectorSubcoreMesh(core_axis_name="c", subcore_axis_name="s")

    @functools.partial(
        pl.kernel,
        mesh=mesh,
        compiler_params=pltpu.CompilerParams(
            use_tc_tiling_on_sc=False, needs_layout_passes=False),
        out_type=jax.ShapeDtypeStruct((N_EDGES, 2 * EMB), jnp.float32),
        scratch_types=[
            [pltpu.VMEM((CHUNK,), jnp.int32) for _ in range(NBUF)],
            [pltpu.VMEM((CHUNK, 2 * EMB), jnp.float32) for _ in range(NBUF)],
            [pltpu.SemaphoreType.DMA for _ in range(NBUF)],
            [pltpu.SemaphoreType.DMA for _ in range(NBUF)],
            [pltpu.SemaphoreType.DMA for _ in range(NBUF)],
        ],
    )
    def k(ci_hbm, wc_hbm, out_hbm, ci_v, out_v, isem, gsem, wsem):
        wid = lax.axis_index("s") * 2 + lax.axis_index("c")
        steps = (NUM_CHUNKS + NW - 1) // NW
        tw = (NUM_CHUNKS - wid + NW - 1) // NW
        my_wc = wc_hbm.at[wid]

        def start_idx(t, b):
            base = (wid + t * NW) * CHUNK
            pltpu.async_copy(ci_hbm.at[pl.ds(base, CHUNK)], ci_v[b], isem[b])

        def wait_idx(b):
            pltpu.make_async_copy(
                ci_hbm.at[pl.ds(0, CHUNK)], ci_v[b], isem[b]).wait()

        def wait_write(b):
            pltpu.make_async_copy(
                out_v[b], out_hbm.at[pl.ds(0, CHUNK), :], wsem[b]).wait()

        def run_chunk(t, b):
            wait_idx(b)
            cps = []
            for j in range(0, CHUNK, 128):
                cps.append(pltpu.async_copy(
                    my_wc.at[ci_v[b].at[pl.ds(j, 128)]],
                    out_v[b].at[pl.ds(j, 128), :], gsem[b]))
            for cp in cps:
                cp.wait()
            base = (wid + t * NW) * CHUNK
            pltpu.async_copy(out_v[b], out_hbm.at[pl.ds(base, CHUNK), :], wsem[b])

        start_idx(0, 0)

        def body(t, carry):
            for bb in range(NBUF):
                @pl.when(lax.rem(t, NBUF) == bb)
                def _(bb=bb):
                    @pl.when(t + 1 < tw)
                    def _():
                        start_idx(t + 1, (bb + 1) % NBUF)

                    @pl.when(t < tw)
                    def _():
                        @pl.when(t >= NBUF)
                        def _():
                            wait_write(bb)
                        run_chunk(t, bb)
            return carry

        lax.fori_loop(0, steps, body, 0)

        for kk in range(NBUF):
            tp = tw - 1 - kk
            for bb in range(NBUF):
                @pl.when(jnp.logical_and(tp >= 0, lax.rem(tp, NBUF) == bb))
                def _(bb=bb):
                    wait_write(bb)

    return k(ci, Wc_rep)


def kernel(edge_attr, W0, W1):
    Wc = jnp.concatenate(
        [jnp.repeat(W0, 4, axis=0), jnp.tile(W1, (4, 1))], axis=1)
    Wc_rep = jnp.tile(Wc[None], (NW, 1, 1))
    ci = edge_attr[:, 0] * 4 + edge_attr[:, 1]
    return _sc_lookup(ci, Wc_rep)


# speedup vs baseline: 4.0311x; 1.3640x over previous
import functools

import jax
import jax.numpy as jnp
from jax import lax
from jax.experimental import pallas as pl
from jax.experimental.pallas import tpu as pltpu
from jax.experimental.pallas import tpu_sc as plsc

EMB = 16
N_EDGES = 1600000
CHUNK = 1280
NUM_CHUNKS = N_EDGES // CHUNK
NW = 32
NBUF = 2


def _sc_lookup(ci, Wc_rep):
    mesh = plsc.VectorSubcoreMesh(core_axis_name="c", subcore_axis_name="s")

    @functools.partial(
        pl.kernel,
        mesh=mesh,
        compiler_params=pltpu.CompilerParams(
            use_tc_tiling_on_sc=False, needs_layout_passes=False),
        out_type=jax.ShapeDtypeStruct((N_EDGES, 2 * EMB), jnp.float32),
        scratch_types=[
            [pltpu.VMEM((CHUNK,), jnp.int32) for _ in range(NBUF)],
            [pltpu.VMEM((CHUNK, 2 * EMB), jnp.float32) for _ in range(NBUF)],
            [pltpu.SemaphoreType.DMA for _ in range(NBUF)],
            [pltpu.SemaphoreType.DMA for _ in range(NBUF)],
            [pltpu.SemaphoreType.DMA for _ in range(NBUF)],
            pltpu.VMEM_SHARED((16, 2 * EMB), jnp.float32),
        ],
    )
    def k(ci_hbm, wc_hbm, out_hbm, ci_v, out_v, isem, gsem, wsem, wc_sh):
        wid = lax.axis_index("s") * 2 + lax.axis_index("c")
        steps = (NUM_CHUNKS + NW - 1) // NW
        tw = (NUM_CHUNKS - wid + NW - 1) // NW

        @pl.when(lax.axis_index("s") == 0)
        def _():
            pltpu.sync_copy(wc_hbm.at[0], wc_sh)
        plsc.subcore_barrier()
        my_wc = wc_sh

        def start_idx(t, b):
            base = (wid + t * NW) * CHUNK
            pltpu.async_copy(ci_hbm.at[pl.ds(base, CHUNK)], ci_v[b], isem[b])

        def wait_idx(b):
            pltpu.make_async_copy(
                ci_hbm.at[pl.ds(0, CHUNK)], ci_v[b], isem[b]).wait()

        def wait_write(b):
            pltpu.make_async_copy(
                out_v[b], out_hbm.at[pl.ds(0, CHUNK), :], wsem[b]).wait()

        def run_chunk(t, b):
            wait_idx(b)
            cps = []
            for j in range(0, CHUNK, 128):
                cps.append(pltpu.async_copy(
                    my_wc.at[ci_v[b].at[pl.ds(j, 128)]],
                    out_v[b].at[pl.ds(j, 128), :], gsem[b]))
            for cp in cps:
                cp.wait()
            base = (wid + t * NW) * CHUNK
            pltpu.async_copy(out_v[b], out_hbm.at[pl.ds(base, CHUNK), :], wsem[b])

        start_idx(0, 0)

        def body(t, carry):
            for bb in range(NBUF):
                @pl.when(lax.rem(t, NBUF) == bb)
                def _(bb=bb):
                    @pl.when(t + 1 < tw)
                    def _():
                        start_idx(t + 1, (bb + 1) % NBUF)

                    @pl.when(t < tw)
                    def _():
                        @pl.when(t >= NBUF)
                        def _():
                            wait_write(bb)
                        run_chunk(t, bb)
            return carry

        lax.fori_loop(0, steps, body, 0)

        for kk in range(NBUF):
            tp = tw - 1 - kk
            for bb in range(NBUF):
                @pl.when(jnp.logical_and(tp >= 0, lax.rem(tp, NBUF) == bb))
                def _(bb=bb):
                    wait_write(bb)

    return k(ci, Wc_rep)


def kernel(edge_attr, W0, W1):
    Wc = jnp.concatenate(
        [jnp.repeat(W0, 4, axis=0), jnp.tile(W1, (4, 1))], axis=1)
    Wc_rep = jnp.tile(Wc[None], (NW, 1, 1))
    ci = edge_attr[:, 0] * 4 + edge_attr[:, 1]
    return _sc_lookup(ci, Wc_rep)
